# baseline (speedup 1.0000x reference)
import jax
import jax.numpy as jnp
from jax import lax
from jax.experimental import pallas as pl
from jax.experimental.pallas import tpu as pltpu
from jax.experimental.pallas import tpu_sc as plsc

_BASE = 0x35800000
_RK_MAX = (1 << 28) - 1
_CHUNK = 65536
_NVEC = _CHUNK // 16


def _prep_body(lp_ref, lt_ref, dp_ref, dt_ref, keys_ref, scal_ref):
    dp = dp_ref[...]
    dt = dt_ref[...]
    pos = dt == 1.0
    neg = dt == 0.0
    loss = -jnp.log(jnp.where(pos, dp, 1.0 - dp))
    keys_ref[...] = jnp.where(
        neg, jax.lax.bitcast_convert_type(loss, jnp.int32), jnp.int32(0))
    scal_ref[0] = jnp.sum(jnp.where(pos, loss, 0.0))
    scal_ref[1] = jnp.sum(jnp.where(pos, 1.0, 0.0))
    scal_ref[2] = jnp.sum(jnp.where(neg, 1.0, 0.0))
    lp = lp_ref[...]
    lt = lt_ref[...]
    scal_ref[3] = jnp.mean((lp - lt) ** 2)


def _select_body(keys_hbm, scal_hbm, out_hbm, c1_st, s1_st, c2_st, s2_st,
                 chunk_v, scal_v, cnt_h, sum_h, mcnt, msum,
                 allc4, alls4, accc, accs, out_v):
    sid = lax.axis_index("s")
    lane = lax.iota(jnp.int32, 16)
    ones_i = jnp.ones((16,), jnp.int32)
    zero_i = jnp.zeros((16,), jnp.int32)
    zero_f = jnp.zeros((16,), jnp.float32)

    def lane_at(vec, j):
        return jnp.sum(jnp.where(lane == j, vec, jnp.zeros_like(vec)))

    pltpu.sync_copy(keys_hbm.at[pl.ds(sid * _CHUNK, _CHUNK)], chunk_v)
    pltpu.sync_copy(scal_hbm, scal_v)
    sv = scal_v[...]
    sum_pos = lane_at(sv, 0)
    num_pos = lane_at(sv, 1)
    num_neg = lane_at(sv, 2)
    mse = lane_at(sv, 3)
    k = (jnp.float32(0.7) * num_neg).astype(jnp.int32)
    kf = k.astype(jnp.float32)

    def zero_hists():
        def zb(i, c):
            cnt_h[pl.ds(i * 16, 16)] = zero_i
            sum_h[pl.ds(i * 16, 16)] = zero_f
            return c
        lax.fori_loop(0, 512, zb, 0)

    def scan_pass(shift, pass1, d1):
        def sb(i, c):
            for u in range(4):
                kv = chunk_v[pl.ds(i * 64 + u * 16, 16)]
                rk = jnp.minimum(jnp.maximum(kv - _BASE, 0), _RK_MAX)
                if pass1:
                    act = kv > 0
                else:
                    act = (kv > 0) & ((rk >> 19) == d1)
                dg = (rk >> shift) & 511
                idx = lane * 512 + dg
                plsc.addupdate_scatter(cnt_h, [idx], ones_i, mask=act)
                plsc.addupdate_scatter(
                    sum_h, [idx], plsc.bitcast(kv, jnp.float32), mask=act)
            return c
        lax.fori_loop(0, _NVEC // 4, sb, 0)

    def fold_local():
        def mb(g, c):
            acc_c = zero_i
            acc_s = zero_f
            for l in range(16):
                acc_c = acc_c + cnt_h[pl.ds(l * 512 + g * 16, 16)]
                acc_s = acc_s + sum_h[pl.ds(l * 512 + g * 16, 16)]
            mcnt[g] = acc_c
            msum[g] = acc_s
            return c
        lax.fori_loop(0, 32, mb, 0)

    def merge_global(c_st, s_st):
        pltpu.sync_copy(mcnt, c_st.at[sid])
        pltpu.sync_copy(msum, s_st.at[sid])
        plsc.subcore_barrier()
        for p8 in range(8):
            pltpu.sync_copy(c_st.at[pl.ds(p8 * 2, 2)], allc4)
            pltpu.sync_copy(s_st.at[pl.ds(p8 * 2, 2)], alls4)
            if p8 == 0:
                def fc0(g, c):
                    accc[g] = allc4[0, g] + allc4[1, g]
                    accs[g] = alls4[0, g] + alls4[1, g]
                    return c
                lax.fori_loop(0, 32, fc0, 0)
            else:
                def fc(g, c):
                    accc[g] = accc[g] + allc4[0, g] + allc4[1, g]
                    accs[g] = accs[g] + alls4[0, g] + alls4[1, g]
                    return c
                lax.fori_loop(0, 32, fc, 0)

    def choose(k_rem):
        lo = jnp.int32(0)
        hi = jnp.int32(512)
        cnt_hi = jnp.int32(0)
        sum_hi = jnp.float32(0.0)
        def probe(mid):
            def pb(g, carry):
                ac, as_ = carry
                m = (g * 16 + lane) >= mid
                ac = ac + jnp.sum(jnp.where(m, accc[g], zero_i))
                as_ = as_ + jnp.sum(jnp.where(m, accs[g], zero_f))
                return ac, as_
            return lax.fori_loop(0, 32, pb, (jnp.int32(0), jnp.float32(0.0)))
        for _ in range(9):
            mid = (lo + hi) // 2
            cge, sge = probe(mid)
            ok = cge >= k_rem
            lo = jnp.where(ok, mid, lo)
            hi = jnp.where(ok, hi, mid)
            cnt_hi = jnp.where(ok, cnt_hi, cge)
            sum_hi = jnp.where(ok, sum_hi, sge)
        return lo, cnt_hi, sum_hi

    zero_hists()
    scan_pass(19, True, jnp.int32(0))
    fold_local()
    merge_global(c1_st, s1_st)
    d1, cnt1, sum1 = choose(k)
    k_rem = k - cnt1

    zero_hists()
    scan_pass(10, False, d1)
    fold_local()
    merge_global(c2_st, s2_st)
    d2, cnt2, sum2 = choose(k_rem)
    k_rem2 = (k_rem - cnt2).astype(jnp.float32)

    mid_key = _BASE + (d1 << 19) + (d2 << 10) + 512
    midv = plsc.bitcast(jnp.broadcast_to(mid_key, (16,)), jnp.float32)
    mid = lane_at(midv, 0)
    sum_topk = sum1 + sum2 + k_rem2 * mid
    out_v[...] = jnp.broadcast_to(sum_topk, (16,))

    @pl.when(sid == 0)
    def _write():
        pltpu.sync_copy(out_v, out_hbm)


def kernel(label_p, label_t, denselabel_p, denselabel_t):
    keys, scal = pl.pallas_call(
        _prep_body,
        out_shape=(
            jax.ShapeDtypeStruct((32, 32768), jnp.int32),
            jax.ShapeDtypeStruct((8,), jnp.float32),
        ),
        out_specs=(
            pl.BlockSpec(memory_space=pltpu.VMEM),
            pl.BlockSpec(memory_space=pltpu.SMEM),
        ),
    )(label_p, label_t, denselabel_p, denselabel_t)

    keys_flat = jnp.reshape(keys, (1048576,))
    scal16 = jnp.concatenate([scal, jnp.zeros((8,), jnp.float32)])

    mesh = plsc.VectorSubcoreMesh(
        core_axis_name="c", subcore_axis_name="s", num_cores=1)
    outs = pl.kernel(
        _select_body,
        out_type=(
            jax.ShapeDtypeStruct((16,), jnp.float32),
            jax.ShapeDtypeStruct((16, 32, 16), jnp.int32),
            jax.ShapeDtypeStruct((16, 32, 16), jnp.float32),
            jax.ShapeDtypeStruct((16, 32, 16), jnp.int32),
            jax.ShapeDtypeStruct((16, 32, 16), jnp.float32),
        ),
        mesh=mesh,
        compiler_params=pltpu.CompilerParams(needs_layout_passes=False),
        scratch_types=[
            pltpu.VMEM((_CHUNK,), jnp.int32),
            pltpu.VMEM((16,), jnp.float32),
            pltpu.VMEM((8192,), jnp.int32),
            pltpu.VMEM((8192,), jnp.float32),
            pltpu.VMEM((32, 16), jnp.int32),
            pltpu.VMEM((32, 16), jnp.float32),
            pltpu.VMEM((2, 32, 16), jnp.int32),
            pltpu.VMEM((2, 32, 16), jnp.float32),
            pltpu.VMEM((32, 16), jnp.int32),
            pltpu.VMEM((32, 16), jnp.float32),
            pltpu.VMEM((16,), jnp.float32),
        ],
    )(keys_flat, scal16)

    sum_pos, num_pos, num_neg, mse = scal[0], scal[1], scal[2], scal[3]
    kf = jnp.floor(0.7 * num_neg)
    return mse + sum_pos / num_pos + outs[0][0] / kf


# speedup vs baseline: 1.6061x; 1.6061x over previous
import jax
import jax.numpy as jnp
from jax import lax
from jax.experimental import pallas as pl
from jax.experimental.pallas import tpu as pltpu
from jax.experimental.pallas import tpu_sc as plsc

_BASE = 0x35800000
_RK_MAX = (1 << 28) - 1
_CHUNK = 32768
_NVEC = _CHUNK // 16


def _prep_body(lp_ref, lt_ref, dp_ref, dt_ref, keys_ref, scal_ref):
    dp = dp_ref[...]
    dt = dt_ref[...]

    pos = dt == 1.0
    neg = dt == 0.0

    loss = -jnp.log(jnp.where(pos, dp, 1.0 - dp))

    keys_ref[...] = jnp.where(
        neg, jax.lax.bitcast_convert_type(loss, jnp.int32), jnp.int32(0))

    scal_ref[0] = jnp.sum(jnp.where(pos, loss, 0.0))
    scal_ref[1] = jnp.sum(jnp.where(pos, 1.0, 0.0))
    scal_ref[2] = jnp.sum(jnp.where(neg, 1.0, 0.0))
    lp = lp_ref[...]
    lt = lt_ref[...]
    scal_ref[3] = jnp.mean((lp - lt) ** 2)


def _make_hist_body(shift, pass1):
    def body(keys_hbm, prm_hbm, cnt_out, sum_out,
             chunk_v, prm_v, cnt_h, sum_h, mcnt, msum):
        sid = lax.axis_index("s") * 2 + lax.axis_index("c")
        lane = lax.iota(jnp.int32, 16)
        ones_i = jnp.ones((16,), jnp.int32)
        zero_i = jnp.zeros((16,), jnp.int32)
        zero_f = jnp.zeros((16,), jnp.float32)

        def lane_at(vec, j):
            return jnp.sum(jnp.where(lane == j, vec, jnp.zeros_like(vec)))

        pltpu.sync_copy(keys_hbm.at[pl.ds(sid * _CHUNK, _CHUNK)], chunk_v)
        pltpu.sync_copy(prm_hbm, prm_v)
        d1 = lane_at(prm_v[...], 0)

        def zb(i, c):
            cnt_h[pl.ds(i * 16, 16)] = zero_i
            sum_h[pl.ds(i * 16, 16)] = zero_f
            return c
        lax.fori_loop(0, 512, zb, 0)

        def sb(i, c):
            for u in range(4):
                kv = chunk_v[pl.ds(i * 64 + u * 16, 16)]
                rk = jnp.minimum(jnp.maximum(kv - _BASE, 0), _RK_MAX)
                if pass1:
                    act = kv > 0
                else:
                    act = (kv > 0) & ((rk >> 19) == d1)
                dg = (rk >> shift) & 511
                idx = lane * 512 + dg
                plsc.addupdate_scatter(cnt_h, [idx], ones_i, mask=act)
                plsc.addupdate_scatter(
                    sum_h, [idx], plsc.bitcast(kv, jnp.float32), mask=act)
            return c
        lax.fori_loop(0, _NVEC // 4, sb, 0)

        def mb(g, c):
            acc_c = zero_i
            acc_s = zero_f
            for l in range(16):
                acc_c = acc_c + cnt_h[pl.ds(l * 512 + g * 16, 16)]
                acc_s = acc_s + sum_h[pl.ds(l * 512 + g * 16, 16)]
            mcnt[g] = acc_c
            msum[g] = acc_s
            return c
        lax.fori_loop(0, 32, mb, 0)
        pltpu.sync_copy(mcnt, cnt_out.at[sid])
        pltpu.sync_copy(msum, sum_out.at[sid])
    return body


def _run_hist(keys_flat, prm, shift, pass1):
    mesh = plsc.VectorSubcoreMesh(
        core_axis_name="c", subcore_axis_name="s", num_cores=2)
    return pl.kernel(
        _make_hist_body(shift, pass1),
        out_type=(
            jax.ShapeDtypeStruct((32, 32, 16), jnp.int32),
            jax.ShapeDtypeStruct((32, 32, 16), jnp.float32),
        ),
        mesh=mesh,
        compiler_params=pltpu.CompilerParams(needs_layout_passes=False),
        scratch_types=[
            pltpu.VMEM((_CHUNK,), jnp.int32),
            pltpu.VMEM((16,), jnp.int32),
            pltpu.VMEM((8192,), jnp.int32),
            pltpu.VMEM((8192,), jnp.float32),
            pltpu.VMEM((32, 16), jnp.int32),
            pltpu.VMEM((32, 16), jnp.float32),
        ],
    )(keys_flat, prm)


def _choose(cnt_rows, sum_rows, k_rem):
    merged_c = jnp.sum(cnt_rows.astype(jnp.float32), axis=0, keepdims=True)
    merged_s = jnp.sum(sum_rows, axis=0, keepdims=True)
    dig = lax.broadcasted_iota(jnp.int32, (1, 512), 1)

    lo = jnp.int32(0)
    hi = jnp.int32(512)
    cnt_hi = jnp.float32(0.0)
    sum_hi = jnp.float32(0.0)
    for _ in range(9):
        mid = (lo + hi) // 2
        cge = jnp.sum(jnp.where(dig >= mid, merged_c, 0.0))
        sge = jnp.sum(jnp.where(dig >= mid, merged_s, 0.0))
        ok = cge >= k_rem.astype(jnp.float32)
        lo = jnp.where(ok, mid, lo)
        hi = jnp.where(ok, hi, mid)
        cnt_hi = jnp.where(ok, cnt_hi, cge)
        sum_hi = jnp.where(ok, sum_hi, sge)
    return lo, cnt_hi.astype(jnp.int32), sum_hi


def _choose1_body(cnt_ref, sum_ref, scal_ref, out_ref):
    num_neg = scal_ref[2]
    k = jnp.floor(0.7 * num_neg).astype(jnp.int32)
    d1, cnt_ab, sum_ab = _choose(cnt_ref[...], sum_ref[...], k)
    out_ref[0] = d1.astype(jnp.float32)
    out_ref[1] = (k - cnt_ab).astype(jnp.float32)
    out_ref[2] = sum_ab
    out_ref[3] = k.astype(jnp.float32)


def _final_body(cnt_ref, sum_ref, scal_ref, c1_ref, out_ref):
    d1 = c1_ref[0].astype(jnp.int32)
    k_rem = c1_ref[1].astype(jnp.int32)
    sum1 = c1_ref[2]
    kf = c1_ref[3]

    d2, cnt_ab, sum2 = _choose(cnt_ref[...], sum_ref[...], k_rem)
    k_rem2 = (k_rem - cnt_ab).astype(jnp.float32)

    mid_key = _BASE + (d1 << 19) + (d2 << 10) + 512
    mid = jax.lax.bitcast_convert_type(mid_key, jnp.float32)
    sum_topk = sum1 + sum2 + k_rem2 * mid

    sum_pos = scal_ref[0]
    num_pos = scal_ref[1]
    mse = scal_ref[3]
    out_ref[0, 0] = mse + sum_pos / num_pos + sum_topk / kf


def kernel(label_p, label_t, denselabel_p, denselabel_t):
    keys, scal = pl.pallas_call(
        _prep_body,
        out_shape=(
            jax.ShapeDtypeStruct((32, 32768), jnp.int32),
            jax.ShapeDtypeStruct((8,), jnp.float32),
        ),
        out_specs=(
            pl.BlockSpec(memory_space=pltpu.VMEM),
            pl.BlockSpec(memory_space=pltpu.SMEM),
        ),
    )(label_p, label_t, denselabel_p, denselabel_t)

    keys_flat = jnp.reshape(keys, (1048576,))
    zero_prm = jnp.zeros((16,), jnp.int32)

    cnt1, sum1 = _run_hist(keys_flat, zero_prm, 19, True)

    c1 = pl.pallas_call(
        _choose1_body,
        out_shape=jax.ShapeDtypeStruct((8,), jnp.float32),
        in_specs=[
            pl.BlockSpec(memory_space=pltpu.VMEM),
            pl.BlockSpec(memory_space=pltpu.VMEM),
            pl.BlockSpec(memory_space=pltpu.SMEM),
        ],
        out_specs=pl.BlockSpec(memory_space=pltpu.SMEM),
    )(jnp.reshape(cnt1, (32, 512)), jnp.reshape(sum1, (32, 512)), scal)

    prm2 = jnp.full((16,), c1[0].astype(jnp.int32), jnp.int32)
    cnt2, sum2 = _run_hist(keys_flat, prm2, 10, False)

    out = pl.pallas_call(
        _final_body,
        out_shape=jax.ShapeDtypeStruct((1, 1), jnp.float32),
        in_specs=[
            pl.BlockSpec(memory_space=pltpu.VMEM),
            pl.BlockSpec(memory_space=pltpu.VMEM),
            pl.BlockSpec(memory_space=pltpu.SMEM),
            pl.BlockSpec(memory_space=pltpu.SMEM),
        ],
        out_specs=pl.BlockSpec(memory_space=pltpu.SMEM),
    )(jnp.reshape(cnt2, (32, 512)), jnp.reshape(sum2, (32, 512)), scal, c1)

    return out[0, 0]
